# baseline (speedup 1.0000x reference)
import functools

import jax
import jax.numpy as jnp
from jax import lax
from jax.experimental import pallas as pl
from jax.experimental.pallas import tpu as pltpu
from jax.experimental.pallas import tpu_sc as plsc

C = 26
V = 100000
D = 32
B = 16384

NC = 2
NS = 16
NW = NC * NS

N = B * C
CHUNK = 1664
CHUNKS_PER_W = N // (NW * CHUNK)


def kernel(input, W):
    w_flat = W.reshape(C * V, D)
    idx_flat = input.reshape(N)
    off = jnp.tile(jnp.arange(C, dtype=jnp.int32) * V, CHUNK // C)

    mesh = plsc.VectorSubcoreMesh(core_axis_name="c", subcore_axis_name="s")

    @functools.partial(
        pl.kernel,
        out_type=jax.ShapeDtypeStruct((N, D), jnp.float32),
        mesh=mesh,
        scratch_types=[
            pltpu.VMEM((CHUNK,), jnp.int32),
            pltpu.VMEM((CHUNK,), jnp.int32),
            pltpu.VMEM((CHUNK, D), jnp.float32),
            pltpu.SemaphoreType.DMA,
        ],
        compiler_params=pltpu.CompilerParams(use_tc_tiling_on_sc=False),
    )
    def k(idx_hbm, off_hbm, w_hbm, out_hbm, idx_v, off_v, rows_v, sem):
        wid = lax.axis_index("s") * NC + lax.axis_index("c")
        pltpu.sync_copy(off_hbm, off_v)

        @pl.loop(0, CHUNKS_PER_W)
        def _(t):
            base = (wid * CHUNKS_PER_W + t) * CHUNK
            pltpu.sync_copy(idx_hbm.at[pl.ds(base, CHUNK)], idx_v)

            @pl.loop(0, CHUNK, step=16)
            def _(i):
                sl = pl.ds(i, 16)
                idx_v.at[sl][...] = idx_v.at[sl][...] + off_v.at[sl][...]

            pltpu.async_copy(w_hbm.at[idx_v], rows_v, sem).wait()
            pltpu.sync_copy(rows_v, out_hbm.at[pl.ds(base, CHUNK)])

    out = k(idx_flat, off, w_flat)
    return out.reshape(B, C, D)


# speedup vs baseline: 1.0081x; 1.0081x over previous
import functools

import jax
import jax.numpy as jnp
from jax import lax
from jax.experimental import pallas as pl
from jax.experimental.pallas import tpu as pltpu
from jax.experimental.pallas import tpu_sc as plsc

C = 26
V = 100000
D = 32
B = 16384

NC = 2
NS = 16
NW = NC * NS

N = B * C
PER_W = N // NW
CHUNK = 1664
CHUNKS_PER_W = PER_W // CHUNK


def kernel(input, W):
    w_flat = W.reshape(C * V, D)
    idx_flat = input.reshape(N)
    off = jnp.tile(jnp.arange(C, dtype=jnp.int32) * V, CHUNK // C)

    mesh = plsc.VectorSubcoreMesh(core_axis_name="c", subcore_axis_name="s")

    @functools.partial(
        pl.kernel,
        out_type=jax.ShapeDtypeStruct((N, D), jnp.float32),
        mesh=mesh,
        scratch_types=[
            pltpu.VMEM((PER_W,), jnp.int32),
            pltpu.VMEM((CHUNK,), jnp.int32),
            pltpu.VMEM((2, CHUNK, D), jnp.float32),
            pltpu.SemaphoreType.DMA,
            pltpu.SemaphoreType.DMA,
            pltpu.SemaphoreType.DMA,
            pltpu.SemaphoreType.DMA,
        ],
        compiler_params=pltpu.CompilerParams(use_tc_tiling_on_sc=False),
    )
    def k(idx_hbm, off_hbm, w_hbm, out_hbm, idx_v, off_v, rows_v,
          sg0, sg1, sw0, sw1):
        wid = lax.axis_index("s") * NC + lax.axis_index("c")
        base = wid * PER_W
        pltpu.sync_copy(idx_hbm.at[pl.ds(base, PER_W)], idx_v)
        pltpu.sync_copy(off_hbm, off_v)

        @pl.loop(0, CHUNKS_PER_W)
        def _(c):
            @pl.loop(0, CHUNK, step=16)
            def _(i):
                sl = pl.ds(c * CHUNK + i, 16)
                idx_v.at[sl][...] = idx_v.at[sl][...] + off_v.at[pl.ds(i, 16)][...]

        sgs = (sg0, sg1)
        sws = (sw0, sw1)
        gathers = {}
        writes = {}
        for t in range(CHUNKS_PER_W):
            b = t % 2
            if t >= 2:
                writes[t - 2].wait()
            gathers[t] = pltpu.async_copy(
                w_hbm.at[idx_v.at[pl.ds(t * CHUNK, CHUNK)]],
                rows_v.at[b], sgs[b])
            gathers[t].wait()
            writes[t] = pltpu.async_copy(
                rows_v.at[b],
                out_hbm.at[pl.ds(base + t * CHUNK, CHUNK)], sws[b])
        writes[CHUNKS_PER_W - 2].wait()
        writes[CHUNKS_PER_W - 1].wait()

    out = k(idx_flat, off, w_flat)
    return out.reshape(B, C, D)
